# Initial kernel scaffold; baseline (speedup 1.0000x reference)
import functools

import jax
import jax.numpy as jnp
from jax import lax
from jax.experimental import pallas as pl
from jax.experimental.pallas import tpu as pltpu
from jax.experimental.pallas import tpu_sc as plsc

N = 10000
E = 320000
F = 128

NC = 2
NS = 16
L = 16

N_PAD = 10240
NPT = N_PAD // NS
B = 128
TB = -(-E // (NS * B))
NBLK = NS * TB
E_PAD = NBLK * B

_MESH = plsc.VectorSubcoreMesh(core_axis_name="c", subcore_axis_name="s")


def _hist_body(idx_hbm, out_hbm, idx_v, ones_v, zero_v, hist_sh):
    c = lax.axis_index("c")
    s = lax.axis_index("s")

    def zfill(i, _):
        zero_v[pl.ds(i * L, L)] = jnp.zeros((L,), jnp.float32)
        return 0

    lax.fori_loop(0, NPT // L, zfill, 0)

    def ofill(i, _):
        ones_v[pl.ds(i * L, L)] = jnp.full((L,), 1.0, jnp.float32)
        return 0

    lax.fori_loop(0, B // L, ofill, 0)

    pltpu.sync_copy(zero_v, hist_sh.at[pl.ds(s * NPT, NPT)])
    pltpu.sync_copy(idx_hbm.at[c, pl.ds(s * TB, TB)], idx_v)
    plsc.subcore_barrier()

    def body(j, _):
        pltpu.sync_copy(ones_v, hist_sh.at[idx_v.at[j]], add=True)
        return 0

    lax.fori_loop(0, TB, body, 0)
    plsc.subcore_barrier()
    pltpu.sync_copy(hist_sh.at[pl.ds(s * NPT, NPT)],
                    out_hbm.at[c, pl.ds(s * NPT, NPT)])


_hist_call = pl.kernel(
    _hist_body,
    out_type=jax.ShapeDtypeStruct((2, N_PAD), jnp.float32),
    mesh=_MESH,
    scratch_types=[
        pltpu.VMEM((TB, B), jnp.int32),
        pltpu.VMEM((B,), jnp.float32),
        pltpu.VMEM((NPT,), jnp.float32),
        pltpu.VMEM_SHARED((N_PAD,), jnp.float32),
    ],
)


def _scatter_body(x_hbm, y_hbm, gidx_hbm, sidx_hbm, zeros_hbm, out_hbm,
                  gidx_v, sidx_v, rows_v, acc_sh, sem):
    c = lax.axis_index("c")
    s = lax.axis_index("s")

    pltpu.sync_copy(zeros_hbm.at[pl.ds(s * NPT, NPT)],
                    acc_sh.at[pl.ds(s * NPT, NPT)])
    pltpu.sync_copy(gidx_hbm.at[c, pl.ds(s * TB, TB)], gidx_v)
    pltpu.sync_copy(sidx_hbm.at[c, pl.ds(s * TB, TB)], sidx_v)
    plsc.subcore_barrier()

    def run(table):
        def body(j, _):
            pltpu.async_copy(table.at[gidx_v.at[j]], rows_v, sem).wait()
            pltpu.sync_copy(rows_v, acc_sh.at[sidx_v.at[j]], add=True)
            return 0

        lax.fori_loop(0, TB, body, 0)

    @pl.when(c == 0)
    def _():
        run(x_hbm)

    @pl.when(c == 1)
    def _():
        run(y_hbm)

    plsc.subcore_barrier()
    pltpu.sync_copy(acc_sh.at[pl.ds(s * NPT, NPT)],
                    out_hbm.at[c, pl.ds(s * NPT, NPT)])


_scatter_call = pl.kernel(
    _scatter_body,
    out_type=jax.ShapeDtypeStruct((2, N_PAD, F), jnp.float32),
    mesh=_MESH,
    scratch_types=[
        pltpu.VMEM((TB, B), jnp.int32),
        pltpu.VMEM((TB, B), jnp.int32),
        pltpu.VMEM((B, F), jnp.float32),
        pltpu.VMEM_SHARED((N_PAD, F), jnp.float32),
        pltpu.SemaphoreType.DMA,
    ],
)


_BN = 512


def _scale_body(x_ref, hc_ref, y_ref):
    deg = hc_ref[...] + 1.0
    dinv = 1.0 / jnp.sqrt(deg)
    y_ref[...] = x_ref[...] * dinv


def _scale_call(x_pad, hc_t):
    grid = (N_PAD // _BN,)
    return pl.pallas_call(
        _scale_body,
        grid=grid,
        in_specs=[
            pl.BlockSpec((_BN, F), lambda i: (i, 0)),
            pl.BlockSpec((_BN, 1), lambda i: (i, 0)),
        ],
        out_specs=pl.BlockSpec((_BN, F), lambda i: (i, 0)),
        out_shape=jax.ShapeDtypeStruct((N_PAD, F), jnp.float32),
    )(x_pad, hc_t)


def _mm(a, b):
    return jnp.dot(a, b, precision=lax.Precision.HIGHEST,
                   preferred_element_type=jnp.float32)


def _leaky(v):
    return jnp.where(v >= 0, v, 0.01 * v)


def _dense_body(head_ref, x_ref, nbs_ref, s_ref, hr_ref, hc_ref,
                g1t_ref, g2t_ref, b1t_ref, b2t_ref, wt_ref, r_ref, b_ref,
                hk_ref, out_ref):
    x = x_ref[...]
    nn = hr_ref[...]
    nb = nbs_ref[...] / jnp.maximum(nn, 1.0)
    gamma = _leaky(_mm(x, g1t_ref[...]) + _mm(nb, g2t_ref[...])) + 1.0
    beta = _leaky(_mm(x, b1t_ref[...]) + _mm(nb, b2t_ref[...]))
    out_rel = x + (gamma * r_ref[...] + beta) - nb
    out_ref[...] = out_rel
    dinv = 1.0 / jnp.sqrt(hc_ref[...] + 1.0)
    t = dinv * s_ref[...] + (dinv * dinv) * x
    h_conv = _mm(t, wt_ref[...]) + b_ref[...]
    h_s = _mm(out_rel, wt_ref[...])
    h_nohead = (h_conv + h_s) / (nn + 1.0)
    hk_ref[...] = jnp.where(head_ref[0, 0] != 0, h_conv, h_nohead)


def _dense_call(head_arr, x_pad, nbs, s_agg, hr_t, hc_t,
                g1t, g2t, b1t, b2t, wt, r, b2d):
    grid = (N_PAD // _BN,)
    blk = lambda i: (i, 0)
    cst = lambda i: (0, 0)
    return pl.pallas_call(
        _dense_body,
        grid=grid,
        in_specs=[
            pl.BlockSpec(memory_space=pltpu.SMEM),
            pl.BlockSpec((_BN, F), blk),
            pl.BlockSpec((_BN, F), blk),
            pl.BlockSpec((_BN, F), blk),
            pl.BlockSpec((_BN, 1), blk),
            pl.BlockSpec((_BN, 1), blk),
            pl.BlockSpec((F, F), cst),
            pl.BlockSpec((F, F), cst),
            pl.BlockSpec((F, F), cst),
            pl.BlockSpec((F, F), cst),
            pl.BlockSpec((F, F), cst),
            pl.BlockSpec((1, F), cst),
            pl.BlockSpec((1, F), cst),
        ],
        out_specs=[
            pl.BlockSpec((_BN, F), blk),
            pl.BlockSpec((_BN, F), blk),
        ],
        out_shape=[
            jax.ShapeDtypeStruct((N_PAD, F), jnp.float32),
            jax.ShapeDtypeStruct((N_PAD, F), jnp.float32),
        ],
    )(head_arr, x_pad, nbs, s_agg, hr_t, hc_t, g1t, g2t, b1t, b2t, wt, r, b2d)


def kernel(x, edge_index, head, G1, G2, B1, B2, r, W_gc, b_gc):
    row = edge_index[0]
    col = edge_index[1]
    x_pad = jnp.pad(x, ((0, N_PAD - N), (0, 0)))

    pad = E_PAD - E
    gidx = jnp.stack([col, row])
    gidx = jnp.pad(gidx, ((0, 0), (0, pad))).reshape(2, NBLK, B)
    sidx = jnp.stack([row, col])
    sidx = jnp.pad(sidx, ((0, 0), (0, pad)), constant_values=N)
    sidx = sidx.reshape(2, NBLK, B)

    hists = _hist_call(sidx)
    hr_t = hists[0].reshape(N_PAD, 1)
    hc_t = hists[1].reshape(N_PAD, 1)

    y = _scale_call(x_pad, hc_t)

    zeros = jnp.zeros((N_PAD, F), jnp.float32)
    accs = _scatter_call(x_pad, y, gidx, sidx, zeros)

    head_arr = jnp.asarray(head, jnp.int32).reshape(1, 1)
    hk, outp = _dense_call(head_arr, x_pad, accs[0], accs[1], hr_t, hc_t,
                           G1.T, G2.T, B1.T, B2.T, W_gc.T, r,
                           b_gc.reshape(1, F))
    return hk[:N], outp[:N]


# speedup vs baseline: 9.9274x; 9.9274x over previous
import functools

import jax
import jax.numpy as jnp
from jax import lax
from jax.experimental import pallas as pl
from jax.experimental.pallas import tpu as pltpu
from jax.experimental.pallas import tpu_sc as plsc

N = 10000
E = 320000
F = 128

NC = 2
NS = 16
L = 16

N_PAD = 10240
NPT = N_PAD // NS
B = 128
C = 32
TB = 160
NBLK = NS * TB
E_PAD = NBLK * B

_MESH = plsc.VectorSubcoreMesh(core_axis_name="c", subcore_axis_name="s")


def _hist_body(idx_hbm, out_hbm, idx_v, ones_v, zero_v, hist_sh):
    c = lax.axis_index("c")
    s = lax.axis_index("s")

    def zfill(i, _):
        zero_v[pl.ds(i * L, L)] = jnp.zeros((L,), jnp.float32)
        return 0

    lax.fori_loop(0, NPT // L, zfill, 0)

    def ofill(i, _):
        ones_v[pl.ds(i * L, L)] = jnp.full((L,), 1.0, jnp.float32)
        return 0

    lax.fori_loop(0, B // L, ofill, 0)

    pltpu.sync_copy(zero_v, hist_sh.at[pl.ds(s * NPT, NPT)])
    pltpu.sync_copy(idx_hbm.at[c, s], idx_v)
    plsc.subcore_barrier()

    def body(j, _):
        pltpu.sync_copy(ones_v, hist_sh.at[idx_v.at[j]], add=True)
        return 0

    lax.fori_loop(0, TB, body, 0)
    plsc.subcore_barrier()
    pltpu.sync_copy(hist_sh.at[pl.ds(s * NPT, NPT)],
                    out_hbm.at[c, pl.ds(s * NPT, NPT)])


_hist_call = pl.kernel(
    _hist_body,
    out_type=jax.ShapeDtypeStruct((2, N_PAD), jnp.float32),
    mesh=_MESH,
    scratch_types=[
        pltpu.VMEM((TB, B), jnp.int32),
        pltpu.VMEM((B,), jnp.float32),
        pltpu.VMEM((NPT,), jnp.float32),
        pltpu.VMEM_SHARED((N_PAD,), jnp.float32),
    ],
)


def _scatter_body(x_hbm, y_hbm, gidx_hbm, sidx_hbm, zeros_hbm, out_hbm,
                  gidx_v, sidx_v, rows_v, acc_sh, sem):
    c = lax.axis_index("c")
    s = lax.axis_index("s")

    pltpu.sync_copy(zeros_hbm.at[pl.ds(s * NPT, NPT)],
                    acc_sh.at[pl.ds(s * NPT, NPT)])
    plsc.subcore_barrier()

    def run(table):
        def chunk(k, _):
            pltpu.sync_copy(gidx_hbm.at[c, s, pl.ds(k * C, C)], gidx_v)
            pltpu.sync_copy(sidx_hbm.at[c, s, pl.ds(k * C, C)], sidx_v)

            def body(j, _):
                pltpu.async_copy(table.at[gidx_v.at[j]], rows_v, sem).wait()
                pltpu.sync_copy(rows_v, acc_sh.at[sidx_v.at[j]], add=True)
                return 0

            lax.fori_loop(0, C, body, 0)
            return 0

        lax.fori_loop(0, TB // C, chunk, 0)

    @pl.when(c == 0)
    def _():
        run(x_hbm)

    @pl.when(c == 1)
    def _():
        run(y_hbm)

    plsc.subcore_barrier()
    pltpu.sync_copy(acc_sh.at[pl.ds(s * NPT, NPT)],
                    out_hbm.at[c, pl.ds(s * NPT, NPT)])


_scatter_call = pl.kernel(
    _scatter_body,
    out_type=jax.ShapeDtypeStruct((2, N_PAD, F), jnp.float32),
    mesh=_MESH,
    scratch_types=[
        pltpu.VMEM((C, B), jnp.int32),
        pltpu.VMEM((C, B), jnp.int32),
        pltpu.VMEM((B, F), jnp.float32),
        pltpu.VMEM_SHARED((N_PAD, F), jnp.float32),
        pltpu.SemaphoreType.DMA,
    ],
)


_BN = 512


def _scale_body(x_ref, hc_ref, y_ref):
    deg = hc_ref[...] + 1.0
    dinv = 1.0 / jnp.sqrt(deg)
    y_ref[...] = x_ref[...] * dinv


def _scale_call(x_pad, hc_t):
    grid = (N_PAD // _BN,)
    return pl.pallas_call(
        _scale_body,
        grid=grid,
        in_specs=[
            pl.BlockSpec((_BN, F), lambda i: (i, 0)),
            pl.BlockSpec((_BN, 1), lambda i: (i, 0)),
        ],
        out_specs=pl.BlockSpec((_BN, F), lambda i: (i, 0)),
        out_shape=jax.ShapeDtypeStruct((N_PAD, F), jnp.float32),
    )(x_pad, hc_t)


def _mm(a, b):
    return jnp.dot(a, b, precision=lax.Precision.HIGHEST,
                   preferred_element_type=jnp.float32)


def _leaky(v):
    return jnp.where(v >= 0, v, 0.01 * v)


def _dense_body(head_ref, x_ref, nbs_ref, s_ref, hr_ref, hc_ref,
                g1t_ref, g2t_ref, b1t_ref, b2t_ref, wt_ref, r_ref, b_ref,
                hk_ref, out_ref):
    x = x_ref[...]
    nn = hr_ref[...]
    nb = nbs_ref[...] / jnp.maximum(nn, 1.0)
    gamma = _leaky(_mm(x, g1t_ref[...]) + _mm(nb, g2t_ref[...])) + 1.0
    beta = _leaky(_mm(x, b1t_ref[...]) + _mm(nb, b2t_ref[...]))
    out_rel = x + (gamma * r_ref[...] + beta) - nb
    out_ref[...] = out_rel
    dinv = 1.0 / jnp.sqrt(hc_ref[...] + 1.0)
    t = dinv * s_ref[...] + (dinv * dinv) * x
    h_conv = _mm(t, wt_ref[...]) + b_ref[...]
    h_s = _mm(out_rel, wt_ref[...])
    h_nohead = (h_conv + h_s) / (nn + 1.0)
    hk_ref[...] = jnp.where(head_ref[0, 0] != 0, h_conv, h_nohead)


def _dense_call(head_arr, x_pad, nbs, s_agg, hr_t, hc_t,
                g1t, g2t, b1t, b2t, wt, r, b2d):
    grid = (N_PAD // _BN,)
    blk = lambda i: (i, 0)
    cst = lambda i: (0, 0)
    return pl.pallas_call(
        _dense_body,
        grid=grid,
        in_specs=[
            pl.BlockSpec(memory_space=pltpu.SMEM),
            pl.BlockSpec((_BN, F), blk),
            pl.BlockSpec((_BN, F), blk),
            pl.BlockSpec((_BN, F), blk),
            pl.BlockSpec((_BN, 1), blk),
            pl.BlockSpec((_BN, 1), blk),
            pl.BlockSpec((F, F), cst),
            pl.BlockSpec((F, F), cst),
            pl.BlockSpec((F, F), cst),
            pl.BlockSpec((F, F), cst),
            pl.BlockSpec((F, F), cst),
            pl.BlockSpec((1, F), cst),
            pl.BlockSpec((1, F), cst),
        ],
        out_specs=[
            pl.BlockSpec((_BN, F), blk),
            pl.BlockSpec((_BN, F), blk),
        ],
        out_shape=[
            jax.ShapeDtypeStruct((N_PAD, F), jnp.float32),
            jax.ShapeDtypeStruct((N_PAD, F), jnp.float32),
        ],
    )(head_arr, x_pad, nbs, s_agg, hr_t, hc_t, g1t, g2t, b1t, b2t, wt, r, b2d)


def kernel(x, edge_index, head, G1, G2, B1, B2, r, W_gc, b_gc):
    row = edge_index[0]
    col = edge_index[1]
    x_pad = jnp.pad(x, ((0, N_PAD - N), (0, 0)))

    pad = E_PAD - E
    gidx = jnp.stack([col, row])
    gidx = jnp.pad(gidx, ((0, 0), (0, pad))).reshape(2, NS, TB, B)
    sidx = jnp.stack([row, col])
    sidx = jnp.pad(sidx, ((0, 0), (0, pad)), constant_values=N)
    sidx = sidx.reshape(2, NS, TB, B)

    hists = _hist_call(sidx)
    hr_t = hists[0].reshape(N_PAD, 1)
    hc_t = hists[1].reshape(N_PAD, 1)

    y = _scale_call(x_pad, hc_t)

    zeros = jnp.zeros((N_PAD, F), jnp.float32)
    accs = _scatter_call(x_pad, y, gidx, sidx, zeros)

    head_arr = jnp.asarray(head, jnp.int32).reshape(1, 1)
    hk, outp = _dense_call(head_arr, x_pad, accs[0], accs[1], hr_t, hc_t,
                           G1.T, G2.T, B1.T, B2.T, W_gc.T, r,
                           b_gc.reshape(1, F))
    return hk[:N], outp[:N]


# speedup vs baseline: 10.7876x; 1.0867x over previous
import functools

import jax
import jax.numpy as jnp
from jax import lax
from jax.experimental import pallas as pl
from jax.experimental.pallas import tpu as pltpu
from jax.experimental.pallas import tpu_sc as plsc

N = 10000
E = 320000
F = 128

NC = 2
NS = 16
L = 16

N_PAD = 10240
NPT = N_PAD // NS
B = 64
C = 16
NBUF = 4
TB = 320
NBLK = NS * TB
E_PAD = NBLK * B
HB = 128
HTB = E_PAD // (NS * HB)

_MESH = plsc.VectorSubcoreMesh(core_axis_name="c", subcore_axis_name="s")


def _hist_body(idx_hbm, out_hbm, idx_v, ones_v, zero_v, hist_sh):
    c = lax.axis_index("c")
    s = lax.axis_index("s")

    def zfill(i, _):
        zero_v[pl.ds(i * L, L)] = jnp.zeros((L,), jnp.float32)
        return 0

    lax.fori_loop(0, NPT // L, zfill, 0)

    def ofill(i, _):
        ones_v[pl.ds(i * L, L)] = jnp.full((L,), 1.0, jnp.float32)
        return 0

    lax.fori_loop(0, HB // L, ofill, 0)

    pltpu.sync_copy(zero_v, hist_sh.at[pl.ds(s * NPT, NPT)])
    pltpu.sync_copy(idx_hbm.at[c, s], idx_v)
    plsc.subcore_barrier()

    def body(j, _):
        pltpu.sync_copy(ones_v, hist_sh.at[idx_v.at[j]], add=True)
        return 0

    lax.fori_loop(0, HTB, body, 0)
    plsc.subcore_barrier()
    pltpu.sync_copy(hist_sh.at[pl.ds(s * NPT, NPT)],
                    out_hbm.at[c, pl.ds(s * NPT, NPT)])


_hist_call = pl.kernel(
    _hist_body,
    out_type=jax.ShapeDtypeStruct((2, N_PAD), jnp.float32),
    mesh=_MESH,
    scratch_types=[
        pltpu.VMEM((HTB, HB), jnp.int32),
        pltpu.VMEM((HB,), jnp.float32),
        pltpu.VMEM((NPT,), jnp.float32),
        pltpu.VMEM_SHARED((N_PAD,), jnp.float32),
    ],
)


def _scatter_body(x_hbm, y_hbm, gidx_hbm, sidx_hbm, zeros_hbm, out_hbm,
                  gidx_v, sidx_v, rows_v, acc_sh, gsem, ssem):
    c = lax.axis_index("c")
    s = lax.axis_index("s")

    pltpu.sync_copy(zeros_hbm.at[pl.ds(s * NPT, NPT)],
                    acc_sh.at[pl.ds(s * NPT, NPT)])
    plsc.subcore_barrier()

    def run(table):
        def gather_start(j):
            pltpu.async_copy(table.at[gidx_v.at[j]], rows_v.at[j % NBUF],
                             gsem)

        def gather_wait(j):
            pltpu.make_async_copy(table.at[gidx_v.at[j]],
                                  rows_v.at[j % NBUF], gsem).wait()

        def scatter_start(j):
            pltpu.async_copy(rows_v.at[j % NBUF], acc_sh.at[sidx_v.at[j]],
                             ssem, add=True)

        def scatter_wait(j):
            pltpu.make_async_copy(rows_v.at[j % NBUF],
                                  acc_sh.at[sidx_v.at[j]], ssem).wait()

        def chunk(k, _):
            pltpu.sync_copy(gidx_hbm.at[c, s, pl.ds(k * C, C)], gidx_v)
            pltpu.sync_copy(sidx_hbm.at[c, s, pl.ds(k * C, C)], sidx_v)
            gather_start(0)
            gather_start(1)
            for j in range(C):
                if j + 2 < C:
                    if j - 2 >= 0:
                        scatter_wait(j - 2)
                    gather_start(j + 2)
                gather_wait(j)
                scatter_start(j)
            for j in range(max(0, C - 4), C):
                scatter_wait(j)
            return 0

        lax.fori_loop(0, TB // C, chunk, 0)

    @pl.when(c == 0)
    def _():
        run(x_hbm)

    @pl.when(c == 1)
    def _():
        run(y_hbm)

    plsc.subcore_barrier()
    pltpu.sync_copy(acc_sh.at[pl.ds(s * NPT, NPT)],
                    out_hbm.at[c, pl.ds(s * NPT, NPT)])


_scatter_call = pl.kernel(
    _scatter_body,
    out_type=jax.ShapeDtypeStruct((2, N_PAD, F), jnp.float32),
    mesh=_MESH,
    scratch_types=[
        pltpu.VMEM((C, B), jnp.int32),
        pltpu.VMEM((C, B), jnp.int32),
        pltpu.VMEM((NBUF, B, F), jnp.float32),
        pltpu.VMEM_SHARED((N_PAD, F), jnp.float32),
        pltpu.SemaphoreType.DMA,
        pltpu.SemaphoreType.DMA,
    ],
)


_BN = 512


def _scale_body(x_ref, hc_ref, y_ref):
    deg = hc_ref[...] + 1.0
    dinv = 1.0 / jnp.sqrt(deg)
    y_ref[...] = x_ref[...] * dinv


def _scale_call(x_pad, hc_t):
    grid = (N_PAD // _BN,)
    return pl.pallas_call(
        _scale_body,
        grid=grid,
        in_specs=[
            pl.BlockSpec((_BN, F), lambda i: (i, 0)),
            pl.BlockSpec((_BN, 1), lambda i: (i, 0)),
        ],
        out_specs=pl.BlockSpec((_BN, F), lambda i: (i, 0)),
        out_shape=jax.ShapeDtypeStruct((N_PAD, F), jnp.float32),
    )(x_pad, hc_t)


def _mm(a, b):
    return jnp.dot(a, b, precision=lax.Precision.HIGHEST,
                   preferred_element_type=jnp.float32)


def _leaky(v):
    return jnp.where(v >= 0, v, 0.01 * v)


def _dense_body(head_ref, x_ref, nbs_ref, s_ref, hr_ref, hc_ref,
                g1t_ref, g2t_ref, b1t_ref, b2t_ref, wt_ref, r_ref, b_ref,
                hk_ref, out_ref):
    x = x_ref[...]
    nn = hr_ref[...]
    nb = nbs_ref[...] / jnp.maximum(nn, 1.0)
    gamma = _leaky(_mm(x, g1t_ref[...]) + _mm(nb, g2t_ref[...])) + 1.0
    beta = _leaky(_mm(x, b1t_ref[...]) + _mm(nb, b2t_ref[...]))
    out_rel = x + (gamma * r_ref[...] + beta) - nb
    out_ref[...] = out_rel
    dinv = 1.0 / jnp.sqrt(hc_ref[...] + 1.0)
    t = dinv * s_ref[...] + (dinv * dinv) * x
    h_conv = _mm(t, wt_ref[...]) + b_ref[...]
    h_s = _mm(out_rel, wt_ref[...])
    h_nohead = (h_conv + h_s) / (nn + 1.0)
    hk_ref[...] = jnp.where(head_ref[0, 0] != 0, h_conv, h_nohead)


def _dense_call(head_arr, x_pad, nbs, s_agg, hr_t, hc_t,
                g1t, g2t, b1t, b2t, wt, r, b2d):
    grid = (N_PAD // _BN,)
    blk = lambda i: (i, 0)
    cst = lambda i: (0, 0)
    return pl.pallas_call(
        _dense_body,
        grid=grid,
        in_specs=[
            pl.BlockSpec(memory_space=pltpu.SMEM),
            pl.BlockSpec((_BN, F), blk),
            pl.BlockSpec((_BN, F), blk),
            pl.BlockSpec((_BN, F), blk),
            pl.BlockSpec((_BN, 1), blk),
            pl.BlockSpec((_BN, 1), blk),
            pl.BlockSpec((F, F), cst),
            pl.BlockSpec((F, F), cst),
            pl.BlockSpec((F, F), cst),
            pl.BlockSpec((F, F), cst),
            pl.BlockSpec((F, F), cst),
            pl.BlockSpec((1, F), cst),
            pl.BlockSpec((1, F), cst),
        ],
        out_specs=[
            pl.BlockSpec((_BN, F), blk),
            pl.BlockSpec((_BN, F), blk),
        ],
        out_shape=[
            jax.ShapeDtypeStruct((N_PAD, F), jnp.float32),
            jax.ShapeDtypeStruct((N_PAD, F), jnp.float32),
        ],
    )(head_arr, x_pad, nbs, s_agg, hr_t, hc_t, g1t, g2t, b1t, b2t, wt, r, b2d)


def kernel(x, edge_index, head, G1, G2, B1, B2, r, W_gc, b_gc):
    row = edge_index[0]
    col = edge_index[1]
    x_pad = jnp.pad(x, ((0, N_PAD - N), (0, 0)))

    pad = E_PAD - E
    gidx = jnp.stack([col, row])
    gidx = jnp.pad(gidx, ((0, 0), (0, pad))).reshape(2, NS, TB, B)
    sidx = jnp.stack([row, col])
    sidx = jnp.pad(sidx, ((0, 0), (0, pad)), constant_values=N)
    sidx_h = sidx.reshape(2, NS, HTB, HB)
    sidx = sidx.reshape(2, NS, TB, B)

    hists = _hist_call(sidx_h)
    hr_t = hists[0].reshape(N_PAD, 1)
    hc_t = hists[1].reshape(N_PAD, 1)

    y = _scale_call(x_pad, hc_t)

    zeros = jnp.zeros((N_PAD, F), jnp.float32)
    accs = _scatter_call(x_pad, y, gidx, sidx, zeros)

    head_arr = jnp.asarray(head, jnp.int32).reshape(1, 1)
    hk, outp = _dense_call(head_arr, x_pad, accs[0], accs[1], hr_t, hc_t,
                           G1.T, G2.T, B1.T, B2.T, W_gc.T, r,
                           b_gc.reshape(1, F))
    return hk[:N], outp[:N]


# speedup vs baseline: 11.5938x; 1.0747x over previous
import functools

import jax
import jax.numpy as jnp
from jax import lax
from jax.experimental import pallas as pl
from jax.experimental.pallas import tpu as pltpu
from jax.experimental.pallas import tpu_sc as plsc

N = 10000
E = 320000
F = 128

NC = 2
NS = 16
L = 16

N_PAD = 10240
NPT = N_PAD // NS
B = 128
C = 16
NBUF = 2
TB = 160
NBLK = NS * TB
E_PAD = NBLK * B
HB = 128
HTB = E_PAD // (NS * HB)

_MESH = plsc.VectorSubcoreMesh(core_axis_name="c", subcore_axis_name="s")


def _hist_body(idx_hbm, out_hbm, idx_v, ones_v, zero_v, hist_sh):
    c = lax.axis_index("c")
    s = lax.axis_index("s")

    def zfill(i, _):
        zero_v[pl.ds(i * L, L)] = jnp.zeros((L,), jnp.float32)
        return 0

    lax.fori_loop(0, NPT // L, zfill, 0)

    def ofill(i, _):
        ones_v[pl.ds(i * L, L)] = jnp.full((L,), 1.0, jnp.float32)
        return 0

    lax.fori_loop(0, HB // L, ofill, 0)

    pltpu.sync_copy(zero_v, hist_sh.at[pl.ds(s * NPT, NPT)])
    pltpu.sync_copy(idx_hbm.at[c, s], idx_v)
    plsc.subcore_barrier()

    def body(j, _):
        pltpu.sync_copy(ones_v, hist_sh.at[idx_v.at[j]], add=True)
        return 0

    lax.fori_loop(0, HTB, body, 0)
    plsc.subcore_barrier()
    pltpu.sync_copy(hist_sh.at[pl.ds(s * NPT, NPT)],
                    out_hbm.at[c, pl.ds(s * NPT, NPT)])


_hist_call = pl.kernel(
    _hist_body,
    out_type=jax.ShapeDtypeStruct((2, N_PAD), jnp.float32),
    mesh=_MESH,
    scratch_types=[
        pltpu.VMEM((HTB, HB), jnp.int32),
        pltpu.VMEM((HB,), jnp.float32),
        pltpu.VMEM((NPT,), jnp.float32),
        pltpu.VMEM_SHARED((N_PAD,), jnp.float32),
    ],
)


def _scatter_body(x_hbm, y_hbm, gidx_hbm, sidx_hbm, zeros_hbm, out_hbm,
                  gidx_v, sidx_v, rows_v, acc_sh, gsem, ssem):
    c = lax.axis_index("c")
    s = lax.axis_index("s")

    pltpu.sync_copy(zeros_hbm.at[pl.ds(s * NPT, NPT)],
                    acc_sh.at[pl.ds(s * NPT, NPT)])
    plsc.subcore_barrier()

    def run(table):
        def gather_start(j):
            pltpu.async_copy(table.at[gidx_v.at[j]], rows_v.at[j % NBUF],
                             gsem)

        def gather_wait(j):
            pltpu.make_async_copy(table.at[gidx_v.at[j]],
                                  rows_v.at[j % NBUF], gsem).wait()

        def scatter_start(j):
            pltpu.async_copy(rows_v.at[j % NBUF], acc_sh.at[sidx_v.at[j]],
                             ssem, add=True)

        def scatter_wait(j):
            pltpu.make_async_copy(rows_v.at[j % NBUF],
                                  acc_sh.at[sidx_v.at[j]], ssem).wait()

        def chunk(k, _):
            pltpu.sync_copy(gidx_hbm.at[c, s, pl.ds(k * C, C)], gidx_v)
            pltpu.sync_copy(sidx_hbm.at[c, s, pl.ds(k * C, C)], sidx_v)
            D = NBUF - 1
            for i in range(min(D, C)):
                gather_start(i)
            for j in range(C):
                i = j + D
                if i < C:
                    if i - NBUF >= 0:
                        scatter_wait(i - NBUF)
                    gather_start(i)
                gather_wait(j)
                scatter_start(j)
            for j in range(max(0, C - NBUF), C):
                scatter_wait(j)
            return 0

        lax.fori_loop(0, TB // C, chunk, 0)

    @pl.when(c == 0)
    def _():
        run(x_hbm)

    @pl.when(c == 1)
    def _():
        run(y_hbm)

    plsc.subcore_barrier()
    pltpu.sync_copy(acc_sh.at[pl.ds(s * NPT, NPT)],
                    out_hbm.at[c, pl.ds(s * NPT, NPT)])


_scatter_call = pl.kernel(
    _scatter_body,
    out_type=jax.ShapeDtypeStruct((2, N_PAD, F), jnp.float32),
    mesh=_MESH,
    scratch_types=[
        pltpu.VMEM((C, B), jnp.int32),
        pltpu.VMEM((C, B), jnp.int32),
        pltpu.VMEM((NBUF, B, F), jnp.float32),
        pltpu.VMEM_SHARED((N_PAD, F), jnp.float32),
        pltpu.SemaphoreType.DMA,
        pltpu.SemaphoreType.DMA,
    ],
)


_BN = 512


def _scale_body(x_ref, hc_ref, y_ref):
    deg = hc_ref[...] + 1.0
    dinv = 1.0 / jnp.sqrt(deg)
    y_ref[...] = x_ref[...] * dinv


def _scale_call(x_pad, hc_t):
    grid = (N_PAD // _BN,)
    return pl.pallas_call(
        _scale_body,
        grid=grid,
        in_specs=[
            pl.BlockSpec((_BN, F), lambda i: (i, 0)),
            pl.BlockSpec((_BN, 1), lambda i: (i, 0)),
        ],
        out_specs=pl.BlockSpec((_BN, F), lambda i: (i, 0)),
        out_shape=jax.ShapeDtypeStruct((N_PAD, F), jnp.float32),
    )(x_pad, hc_t)


def _mm(a, b):
    return jnp.dot(a, b, precision=lax.Precision.HIGHEST,
                   preferred_element_type=jnp.float32)


def _leaky(v):
    return jnp.where(v >= 0, v, 0.01 * v)


def _dense_body(head_ref, x_ref, nbs_ref, s_ref, hr_ref, hc_ref,
                g1t_ref, g2t_ref, b1t_ref, b2t_ref, wt_ref, r_ref, b_ref,
                hk_ref, out_ref):
    x = x_ref[...]
    nn = hr_ref[...]
    nb = nbs_ref[...] / jnp.maximum(nn, 1.0)
    gamma = _leaky(_mm(x, g1t_ref[...]) + _mm(nb, g2t_ref[...])) + 1.0
    beta = _leaky(_mm(x, b1t_ref[...]) + _mm(nb, b2t_ref[...]))
    out_rel = x + (gamma * r_ref[...] + beta) - nb
    out_ref[...] = out_rel
    dinv = 1.0 / jnp.sqrt(hc_ref[...] + 1.0)
    t = dinv * s_ref[...] + (dinv * dinv) * x
    h_conv = _mm(t, wt_ref[...]) + b_ref[...]
    h_s = _mm(out_rel, wt_ref[...])
    h_nohead = (h_conv + h_s) / (nn + 1.0)
    hk_ref[...] = jnp.where(head_ref[0, 0] != 0, h_conv, h_nohead)


def _dense_call(head_arr, x_pad, nbs, s_agg, hr_t, hc_t,
                g1t, g2t, b1t, b2t, wt, r, b2d):
    grid = (N_PAD // _BN,)
    blk = lambda i: (i, 0)
    cst = lambda i: (0, 0)
    return pl.pallas_call(
        _dense_body,
        grid=grid,
        in_specs=[
            pl.BlockSpec(memory_space=pltpu.SMEM),
            pl.BlockSpec((_BN, F), blk),
            pl.BlockSpec((_BN, F), blk),
            pl.BlockSpec((_BN, F), blk),
            pl.BlockSpec((_BN, 1), blk),
            pl.BlockSpec((_BN, 1), blk),
            pl.BlockSpec((F, F), cst),
            pl.BlockSpec((F, F), cst),
            pl.BlockSpec((F, F), cst),
            pl.BlockSpec((F, F), cst),
            pl.BlockSpec((F, F), cst),
            pl.BlockSpec((1, F), cst),
            pl.BlockSpec((1, F), cst),
        ],
        out_specs=[
            pl.BlockSpec((_BN, F), blk),
            pl.BlockSpec((_BN, F), blk),
        ],
        out_shape=[
            jax.ShapeDtypeStruct((N_PAD, F), jnp.float32),
            jax.ShapeDtypeStruct((N_PAD, F), jnp.float32),
        ],
    )(head_arr, x_pad, nbs, s_agg, hr_t, hc_t, g1t, g2t, b1t, b2t, wt, r, b2d)


def kernel(x, edge_index, head, G1, G2, B1, B2, r, W_gc, b_gc):
    row = edge_index[0]
    col = edge_index[1]
    x_pad = jnp.pad(x, ((0, N_PAD - N), (0, 0)))

    pad = E_PAD - E
    gidx = jnp.stack([col, row])
    gidx = jnp.pad(gidx, ((0, 0), (0, pad))).reshape(2, NS, TB, B)
    sidx = jnp.stack([row, col])
    sidx = jnp.pad(sidx, ((0, 0), (0, pad)), constant_values=N)
    sidx_h = sidx.reshape(2, NS, HTB, HB)
    sidx = sidx.reshape(2, NS, TB, B)

    hists = _hist_call(sidx_h)
    hr_t = hists[0].reshape(N_PAD, 1)
    hc_t = hists[1].reshape(N_PAD, 1)

    y = _scale_call(x_pad, hc_t)

    zeros = jnp.zeros((N_PAD, F), jnp.float32)
    accs = _scatter_call(x_pad, y, gidx, sidx, zeros)

    head_arr = jnp.asarray(head, jnp.int32).reshape(1, 1)
    hk, outp = _dense_call(head_arr, x_pad, accs[0], accs[1], hr_t, hc_t,
                           G1.T, G2.T, B1.T, B2.T, W_gc.T, r,
                           b_gc.reshape(1, F))
    return hk[:N], outp[:N]


# speedup vs baseline: 16.4856x; 1.4219x over previous
import functools

import jax
import jax.numpy as jnp
from jax import lax
from jax.experimental import pallas as pl
from jax.experimental.pallas import tpu as pltpu
from jax.experimental.pallas import tpu_sc as plsc

N = 10000
E = 320000
F = 128

NC = 2
NS = 16
L = 16

N_PAD = 10240
NPT = N_PAD // NS
B = 128
C = 16
NBUF = 2
TB = 160
NBLK = NS * TB
E_PAD = NBLK * B
HB = 128
HTB = E_PAD // (NS * HB)

_MESH = plsc.VectorSubcoreMesh(core_axis_name="c", subcore_axis_name="s")


def _hist_body(idx_hbm, out_hbm, idx_v, ones_v, zero_v, hist_sh):
    c = lax.axis_index("c")
    s = lax.axis_index("s")

    def zfill(i, _):
        zero_v[pl.ds(i * L, L)] = jnp.zeros((L,), jnp.float32)
        return 0

    lax.fori_loop(0, NPT // L, zfill, 0)

    def ofill(i, _):
        ones_v[pl.ds(i * L, L)] = jnp.full((L,), 1.0, jnp.float32)
        return 0

    lax.fori_loop(0, HB // L, ofill, 0)

    pltpu.sync_copy(zero_v, hist_sh.at[pl.ds(s * NPT, NPT)])
    pltpu.sync_copy(idx_hbm.at[c, s], idx_v)
    plsc.subcore_barrier()

    def body(j, _):
        pltpu.sync_copy(ones_v, hist_sh.at[idx_v.at[j]], add=True)
        return 0

    lax.fori_loop(0, HTB, body, 0)
    plsc.subcore_barrier()
    pltpu.sync_copy(hist_sh.at[pl.ds(s * NPT, NPT)],
                    out_hbm.at[c, pl.ds(s * NPT, NPT)])


_hist_call = pl.kernel(
    _hist_body,
    out_type=jax.ShapeDtypeStruct((2, N_PAD), jnp.float32),
    mesh=_MESH,
    scratch_types=[
        pltpu.VMEM((HTB, HB), jnp.int32),
        pltpu.VMEM((HB,), jnp.float32),
        pltpu.VMEM((NPT,), jnp.float32),
        pltpu.VMEM_SHARED((N_PAD,), jnp.float32),
    ],
)


def _scatter_body(x_hbm, y_hbm, gidx_hbm, sidx_hbm, zeros_hbm, out_hbm,
                  gidx_v, sidx_v, rows_v, acc_sh, gsem, ssem):
    c = lax.axis_index("c")
    s = lax.axis_index("s")

    pltpu.sync_copy(zeros_hbm.at[pl.ds(s * NPT, NPT)],
                    acc_sh.at[pl.ds(s * NPT, NPT)])
    plsc.subcore_barrier()

    def run(table):
        def gather_start(j):
            pltpu.async_copy(table.at[gidx_v.at[j]], rows_v.at[j % NBUF],
                             gsem)

        def gather_wait(j):
            pltpu.make_async_copy(table.at[gidx_v.at[j]],
                                  rows_v.at[j % NBUF], gsem).wait()

        def scatter_start(j):
            pltpu.async_copy(rows_v.at[j % NBUF], acc_sh.at[sidx_v.at[j]],
                             ssem, add=True)

        def scatter_wait(j):
            pltpu.make_async_copy(rows_v.at[j % NBUF],
                                  acc_sh.at[sidx_v.at[j]], ssem).wait()

        def chunk(k, _):
            pltpu.sync_copy(gidx_hbm.at[c, s, pl.ds(k * C, C)], gidx_v)
            pltpu.sync_copy(sidx_hbm.at[c, s, pl.ds(k * C, C)], sidx_v)
            D = NBUF - 1
            for i in range(min(D, C)):
                gather_start(i)
            for j in range(C):
                i = j + D
                if i < C:
                    if i - NBUF >= 0:
                        scatter_wait(i - NBUF)
                    gather_start(i)
                gather_wait(j)
                scatter_start(j)
            for j in range(max(0, C - NBUF), C):
                scatter_wait(j)
            return 0

        lax.fori_loop(0, TB // C, chunk, 0)

    @pl.when(c == 0)
    def _():
        run(x_hbm)

    @pl.when(c == 1)
    def _():
        run(y_hbm)

    plsc.subcore_barrier()
    pltpu.sync_copy(acc_sh.at[pl.ds(s * NPT, NPT)],
                    out_hbm.at[c, pl.ds(s * NPT, NPT)])


_scatter_call = pl.kernel(
    _scatter_body,
    out_type=jax.ShapeDtypeStruct((2, N_PAD, F), jnp.bfloat16),
    mesh=_MESH,
    compiler_params=pltpu.CompilerParams(use_tc_tiling_on_sc=False),
    scratch_types=[
        pltpu.VMEM((C, B), jnp.int32),
        pltpu.VMEM((C, B), jnp.int32),
        pltpu.VMEM((NBUF, B, F), jnp.bfloat16),
        pltpu.VMEM_SHARED((N_PAD, F), jnp.bfloat16),
        pltpu.SemaphoreType.DMA,
        pltpu.SemaphoreType.DMA,
    ],
)


_BN = 512


def _scale_body(x_ref, hc_ref, y_ref, xb_ref):
    deg = hc_ref[...] + 1.0
    dinv = 1.0 / jnp.sqrt(deg)
    y_ref[...] = (x_ref[...] * dinv).astype(jnp.bfloat16)
    xb_ref[...] = x_ref[...].astype(jnp.bfloat16)


def _scale_call(x_pad, hc_t):
    grid = (N_PAD // _BN,)
    return pl.pallas_call(
        _scale_body,
        grid=grid,
        in_specs=[
            pl.BlockSpec((_BN, F), lambda i: (i, 0)),
            pl.BlockSpec((_BN, 1), lambda i: (i, 0)),
        ],
        out_specs=[
            pl.BlockSpec((_BN, F), lambda i: (i, 0)),
            pl.BlockSpec((_BN, F), lambda i: (i, 0)),
        ],
        out_shape=[
            jax.ShapeDtypeStruct((N_PAD, F), jnp.bfloat16),
            jax.ShapeDtypeStruct((N_PAD, F), jnp.bfloat16),
        ],
    )(x_pad, hc_t)


def _mm(a, b):
    return jnp.dot(a, b, precision=lax.Precision.HIGHEST,
                   preferred_element_type=jnp.float32)


def _leaky(v):
    return jnp.where(v >= 0, v, 0.01 * v)


def _dense_body(head_ref, x_ref, nbs_ref, s_ref, hr_ref, hc_ref,
                g1t_ref, g2t_ref, b1t_ref, b2t_ref, wt_ref, r_ref, b_ref,
                hk_ref, out_ref):
    x = x_ref[...]
    nn = hr_ref[...]
    nb = nbs_ref[...].astype(jnp.float32) / jnp.maximum(nn, 1.0)
    gamma = _leaky(_mm(x, g1t_ref[...]) + _mm(nb, g2t_ref[...])) + 1.0
    beta = _leaky(_mm(x, b1t_ref[...]) + _mm(nb, b2t_ref[...]))
    out_rel = x + (gamma * r_ref[...] + beta) - nb
    out_ref[...] = out_rel
    dinv = 1.0 / jnp.sqrt(hc_ref[...] + 1.0)
    t = dinv * s_ref[...].astype(jnp.float32) + (dinv * dinv) * x
    h_conv = _mm(t, wt_ref[...]) + b_ref[...]
    h_s = _mm(out_rel, wt_ref[...])
    h_nohead = (h_conv + h_s) / (nn + 1.0)
    hk_ref[...] = jnp.where(head_ref[0, 0] != 0, h_conv, h_nohead)


def _dense_call(head_arr, x_pad, nbs, s_agg, hr_t, hc_t,
                g1t, g2t, b1t, b2t, wt, r, b2d):
    grid = (N_PAD // _BN,)
    blk = lambda i: (i, 0)
    cst = lambda i: (0, 0)
    return pl.pallas_call(
        _dense_body,
        grid=grid,
        in_specs=[
            pl.BlockSpec(memory_space=pltpu.SMEM),
            pl.BlockSpec((_BN, F), blk),
            pl.BlockSpec((_BN, F), blk),
            pl.BlockSpec((_BN, F), blk),
            pl.BlockSpec((_BN, 1), blk),
            pl.BlockSpec((_BN, 1), blk),
            pl.BlockSpec((F, F), cst),
            pl.BlockSpec((F, F), cst),
            pl.BlockSpec((F, F), cst),
            pl.BlockSpec((F, F), cst),
            pl.BlockSpec((F, F), cst),
            pl.BlockSpec((1, F), cst),
            pl.BlockSpec((1, F), cst),
        ],
        out_specs=[
            pl.BlockSpec((_BN, F), blk),
            pl.BlockSpec((_BN, F), blk),
        ],
        out_shape=[
            jax.ShapeDtypeStruct((N_PAD, F), jnp.float32),
            jax.ShapeDtypeStruct((N_PAD, F), jnp.float32),
        ],
    )(head_arr, x_pad, nbs, s_agg, hr_t, hc_t, g1t, g2t, b1t, b2t, wt, r, b2d)


def kernel(x, edge_index, head, G1, G2, B1, B2, r, W_gc, b_gc):
    row = edge_index[0]
    col = edge_index[1]
    x_pad = jnp.pad(x, ((0, N_PAD - N), (0, 0)))

    pad = E_PAD - E
    gidx = jnp.stack([col, row])
    gidx = jnp.pad(gidx, ((0, 0), (0, pad))).reshape(2, NS, TB, B)
    sidx = jnp.stack([row, col])
    sidx = jnp.pad(sidx, ((0, 0), (0, pad)), constant_values=N)
    sidx_h = sidx.reshape(2, NS, HTB, HB)
    sidx = sidx.reshape(2, NS, TB, B)

    hists = _hist_call(sidx_h)
    hr_t = hists[0].reshape(N_PAD, 1)
    hc_t = hists[1].reshape(N_PAD, 1)

    y_b, x_b = _scale_call(x_pad, hc_t)

    zeros = jnp.zeros((N_PAD, F), jnp.bfloat16)
    accs = _scatter_call(x_b, y_b, gidx, sidx, zeros)

    head_arr = jnp.asarray(head, jnp.int32).reshape(1, 1)
    hk, outp = _dense_call(head_arr, x_pad, accs[0], accs[1], hr_t, hc_t,
                           G1.T, G2.T, B1.T, B2.T, W_gc.T, r,
                           b_gc.reshape(1, F))
    return hk[:N], outp[:N]


# speedup vs baseline: 17.0889x; 1.0366x over previous
import functools

import jax
import jax.numpy as jnp
from jax import lax
from jax.experimental import pallas as pl
from jax.experimental.pallas import tpu as pltpu
from jax.experimental.pallas import tpu_sc as plsc

N = 10000
E = 320000
F = 128

NC = 2
NS = 16
L = 16

N_PAD = 10240
NPT = N_PAD // NS
B = 128
C = 32
NBUF = 4
TB = 160
NBLK = NS * TB
E_PAD = NBLK * B
HB = 128
HTB = E_PAD // (NS * HB)

_MESH = plsc.VectorSubcoreMesh(core_axis_name="c", subcore_axis_name="s")


def _hist_body(idx_hbm, out_hbm, idx_v, ones_v, zero_v, hist_sh):
    c = lax.axis_index("c")
    s = lax.axis_index("s")

    def zfill(i, _):
        zero_v[pl.ds(i * L, L)] = jnp.zeros((L,), jnp.float32)
        return 0

    lax.fori_loop(0, NPT // L, zfill, 0)

    def ofill(i, _):
        ones_v[pl.ds(i * L, L)] = jnp.full((L,), 1.0, jnp.float32)
        return 0

    lax.fori_loop(0, HB // L, ofill, 0)

    pltpu.sync_copy(zero_v, hist_sh.at[pl.ds(s * NPT, NPT)])
    pltpu.sync_copy(idx_hbm.at[c, s], idx_v)
    plsc.subcore_barrier()

    def body(j, _):
        pltpu.sync_copy(ones_v, hist_sh.at[idx_v.at[j]], add=True)
        return 0

    lax.fori_loop(0, HTB, body, 0)
    plsc.subcore_barrier()
    pltpu.sync_copy(hist_sh.at[pl.ds(s * NPT, NPT)],
                    out_hbm.at[c, pl.ds(s * NPT, NPT)])


_hist_call = pl.kernel(
    _hist_body,
    out_type=jax.ShapeDtypeStruct((2, N_PAD), jnp.float32),
    mesh=_MESH,
    scratch_types=[
        pltpu.VMEM((HTB, HB), jnp.int32),
        pltpu.VMEM((HB,), jnp.float32),
        pltpu.VMEM((NPT,), jnp.float32),
        pltpu.VMEM_SHARED((N_PAD,), jnp.float32),
    ],
)


def _scatter_body(x_hbm, y_hbm, gidx_hbm, sidx_hbm, zeros_hbm, out_hbm,
                  gidx_v, sidx_v, rows_v, acc_sh, gsem, ssem):
    c = lax.axis_index("c")
    s = lax.axis_index("s")

    pltpu.sync_copy(zeros_hbm.at[pl.ds(s * NPT, NPT)],
                    acc_sh.at[pl.ds(s * NPT, NPT)])
    plsc.subcore_barrier()

    def run(table):
        def gather_start(j):
            pltpu.async_copy(table.at[gidx_v.at[j]], rows_v.at[j % NBUF],
                             gsem)

        def gather_wait(j):
            pltpu.make_async_copy(table.at[gidx_v.at[j]],
                                  rows_v.at[j % NBUF], gsem).wait()

        def scatter_start(j):
            pltpu.async_copy(rows_v.at[j % NBUF], acc_sh.at[sidx_v.at[j]],
                             ssem, add=True)

        def scatter_wait(j):
            pltpu.make_async_copy(rows_v.at[j % NBUF],
                                  acc_sh.at[sidx_v.at[j]], ssem).wait()

        def chunk(k, _):
            pltpu.sync_copy(gidx_hbm.at[c, s, pl.ds(k * C, C)], gidx_v)
            pltpu.sync_copy(sidx_hbm.at[c, s, pl.ds(k * C, C)], sidx_v)
            D = NBUF - 1
            for i in range(min(D, C)):
                gather_start(i)
            for j in range(C):
                i = j + D
                if i < C:
                    if i - NBUF >= 0:
                        scatter_wait(i - NBUF)
                    gather_start(i)
                gather_wait(j)
                scatter_start(j)
            for j in range(max(0, C - NBUF), C):
                scatter_wait(j)
            return 0

        lax.fori_loop(0, TB // C, chunk, 0)

    @pl.when(c == 0)
    def _():
        run(x_hbm)

    @pl.when(c == 1)
    def _():
        run(y_hbm)

    plsc.subcore_barrier()
    pltpu.sync_copy(acc_sh.at[pl.ds(s * NPT, NPT)],
                    out_hbm.at[c, pl.ds(s * NPT, NPT)])


_scatter_call = pl.kernel(
    _scatter_body,
    out_type=jax.ShapeDtypeStruct((2, N_PAD, F), jnp.bfloat16),
    mesh=_MESH,
    compiler_params=pltpu.CompilerParams(use_tc_tiling_on_sc=False),
    scratch_types=[
        pltpu.VMEM((C, B), jnp.int32),
        pltpu.VMEM((C, B), jnp.int32),
        pltpu.VMEM((NBUF, B, F), jnp.bfloat16),
        pltpu.VMEM_SHARED((N_PAD, F), jnp.bfloat16),
        pltpu.SemaphoreType.DMA,
        pltpu.SemaphoreType.DMA,
    ],
)


_BN = 512


def _scale_body(x_ref, hc_ref, y_ref, xb_ref):
    deg = hc_ref[...] + 1.0
    dinv = 1.0 / jnp.sqrt(deg)
    y_ref[...] = (x_ref[...] * dinv).astype(jnp.bfloat16)
    xb_ref[...] = x_ref[...].astype(jnp.bfloat16)


def _scale_call(x_pad, hc_t):
    grid = (N_PAD // _BN,)
    return pl.pallas_call(
        _scale_body,
        grid=grid,
        in_specs=[
            pl.BlockSpec((_BN, F), lambda i: (i, 0)),
            pl.BlockSpec((_BN, 1), lambda i: (i, 0)),
        ],
        out_specs=[
            pl.BlockSpec((_BN, F), lambda i: (i, 0)),
            pl.BlockSpec((_BN, F), lambda i: (i, 0)),
        ],
        out_shape=[
            jax.ShapeDtypeStruct((N_PAD, F), jnp.bfloat16),
            jax.ShapeDtypeStruct((N_PAD, F), jnp.bfloat16),
        ],
    )(x_pad, hc_t)


def _mm(a, b):
    return jnp.dot(a, b, precision=lax.Precision.HIGHEST,
                   preferred_element_type=jnp.float32)


def _leaky(v):
    return jnp.where(v >= 0, v, 0.01 * v)


def _dense_body(head_ref, x_ref, nbs_ref, s_ref, hr_ref, hc_ref,
                g1t_ref, g2t_ref, b1t_ref, b2t_ref, wt_ref, r_ref, b_ref,
                hk_ref, out_ref):
    x = x_ref[...]
    nn = hr_ref[...]
    nb = nbs_ref[...].astype(jnp.float32) / jnp.maximum(nn, 1.0)
    gamma = _leaky(_mm(x, g1t_ref[...]) + _mm(nb, g2t_ref[...])) + 1.0
    beta = _leaky(_mm(x, b1t_ref[...]) + _mm(nb, b2t_ref[...]))
    out_rel = x + (gamma * r_ref[...] + beta) - nb
    out_ref[...] = out_rel
    dinv = 1.0 / jnp.sqrt(hc_ref[...] + 1.0)
    t = dinv * s_ref[...].astype(jnp.float32) + (dinv * dinv) * x
    h_conv = _mm(t, wt_ref[...]) + b_ref[...]
    h_s = _mm(out_rel, wt_ref[...])
    h_nohead = (h_conv + h_s) / (nn + 1.0)
    hk_ref[...] = jnp.where(head_ref[0, 0] != 0, h_conv, h_nohead)


def _dense_call(head_arr, x_pad, nbs, s_agg, hr_t, hc_t,
                g1t, g2t, b1t, b2t, wt, r, b2d):
    grid = (N_PAD // _BN,)
    blk = lambda i: (i, 0)
    cst = lambda i: (0, 0)
    return pl.pallas_call(
        _dense_body,
        grid=grid,
        in_specs=[
            pl.BlockSpec(memory_space=pltpu.SMEM),
            pl.BlockSpec((_BN, F), blk),
            pl.BlockSpec((_BN, F), blk),
            pl.BlockSpec((_BN, F), blk),
            pl.BlockSpec((_BN, 1), blk),
            pl.BlockSpec((_BN, 1), blk),
            pl.BlockSpec((F, F), cst),
            pl.BlockSpec((F, F), cst),
            pl.BlockSpec((F, F), cst),
            pl.BlockSpec((F, F), cst),
            pl.BlockSpec((F, F), cst),
            pl.BlockSpec((1, F), cst),
            pl.BlockSpec((1, F), cst),
        ],
        out_specs=[
            pl.BlockSpec((_BN, F), blk),
            pl.BlockSpec((_BN, F), blk),
        ],
        out_shape=[
            jax.ShapeDtypeStruct((N_PAD, F), jnp.float32),
            jax.ShapeDtypeStruct((N_PAD, F), jnp.float32),
        ],
    )(head_arr, x_pad, nbs, s_agg, hr_t, hc_t, g1t, g2t, b1t, b2t, wt, r, b2d)


def kernel(x, edge_index, head, G1, G2, B1, B2, r, W_gc, b_gc):
    row = edge_index[0]
    col = edge_index[1]
    x_pad = jnp.pad(x, ((0, N_PAD - N), (0, 0)))

    pad = E_PAD - E
    gidx = jnp.stack([col, row])
    gidx = jnp.pad(gidx, ((0, 0), (0, pad))).reshape(2, NS, TB, B)
    sidx = jnp.stack([row, col])
    sidx = jnp.pad(sidx, ((0, 0), (0, pad)), constant_values=N)
    sidx_h = sidx.reshape(2, NS, HTB, HB)
    sidx = sidx.reshape(2, NS, TB, B)

    hists = _hist_call(sidx_h)
    hr_t = hists[0].reshape(N_PAD, 1)
    hc_t = hists[1].reshape(N_PAD, 1)

    y_b, x_b = _scale_call(x_pad, hc_t)

    zeros = jnp.zeros((N_PAD, F), jnp.bfloat16)
    accs = _scatter_call(x_b, y_b, gidx, sidx, zeros)

    head_arr = jnp.asarray(head, jnp.int32).reshape(1, 1)
    hk, outp = _dense_call(head_arr, x_pad, accs[0], accs[1], hr_t, hc_t,
                           G1.T, G2.T, B1.T, B2.T, W_gc.T, r,
                           b_gc.reshape(1, F))
    return hk[:N], outp[:N]


# speedup vs baseline: 17.7485x; 1.0386x over previous
import functools

import jax
import jax.numpy as jnp
from jax import lax
from jax.experimental import pallas as pl
from jax.experimental.pallas import tpu as pltpu
from jax.experimental.pallas import tpu_sc as plsc

N = 10000
E = 320000
F = 128

NC = 2
NS = 16
L = 16

N_PAD = 10240
NPT = N_PAD // NS
B = 128
C = 32
NBUF = 4
TB = 160
NBLK = NS * TB
E_PAD = NBLK * B
HB = 128
HTB = E_PAD // (NS * HB)

_MESH = plsc.VectorSubcoreMesh(core_axis_name="c", subcore_axis_name="s")


def _hist_body(idx_hbm, out_hbm, idx_v, ones_v, zero_v, hist_sh):
    c = lax.axis_index("c")
    s = lax.axis_index("s")

    def zfill(i, _):
        zero_v[pl.ds(i * L, L)] = jnp.zeros((L,), jnp.float32)
        return 0

    lax.fori_loop(0, NPT // L, zfill, 0)

    def ofill(i, _):
        ones_v[pl.ds(i * L, L)] = jnp.full((L,), 1.0, jnp.float32)
        return 0

    lax.fori_loop(0, HB // L, ofill, 0)

    pltpu.sync_copy(zero_v, hist_sh.at[pl.ds(s * NPT, NPT)])
    pltpu.sync_copy(idx_hbm.at[c, s], idx_v)
    plsc.subcore_barrier()

    def body(j, _):
        pltpu.sync_copy(ones_v, hist_sh.at[idx_v.at[j]], add=True)
        return 0

    lax.fori_loop(0, HTB, body, 0)
    plsc.subcore_barrier()
    pltpu.sync_copy(hist_sh.at[pl.ds(s * NPT, NPT)],
                    out_hbm.at[c, pl.ds(s * NPT, NPT)])


_hist_call = pl.kernel(
    _hist_body,
    out_type=jax.ShapeDtypeStruct((2, N_PAD), jnp.float32),
    mesh=_MESH,
    scratch_types=[
        pltpu.VMEM((HTB, HB), jnp.int32),
        pltpu.VMEM((HB,), jnp.float32),
        pltpu.VMEM((NPT,), jnp.float32),
        pltpu.VMEM_SHARED((N_PAD,), jnp.float32),
    ],
)


def _scatter_body(x_hbm, y_hbm, gidx_hbm, sidx_hbm, zeros_hbm, out_hbm,
                  gidx_v, sidx_v, rows_v, acc_sh, gsem, ssem):
    c = lax.axis_index("c")
    s = lax.axis_index("s")

    pltpu.sync_copy(zeros_hbm.at[pl.ds(s * NPT, NPT)],
                    acc_sh.at[pl.ds(s * NPT, NPT)])
    plsc.subcore_barrier()

    def run(table):
        def gather_start(j):
            pltpu.async_copy(table.at[gidx_v.at[j]], rows_v.at[j % NBUF],
                             gsem)

        def gather_wait(j):
            pltpu.make_async_copy(table.at[gidx_v.at[j]],
                                  rows_v.at[j % NBUF], gsem).wait()

        def scatter_start(j):
            pltpu.async_copy(rows_v.at[j % NBUF], acc_sh.at[sidx_v.at[j]],
                             ssem, add=True)

        def scatter_wait(j):
            pltpu.make_async_copy(rows_v.at[j % NBUF],
                                  acc_sh.at[sidx_v.at[j]], ssem).wait()

        def chunk(k, _):
            pltpu.sync_copy(gidx_hbm.at[c, s, pl.ds(k * C, C)], gidx_v)
            pltpu.sync_copy(sidx_hbm.at[c, s, pl.ds(k * C, C)], sidx_v)
            D = NBUF - 1
            for i in range(min(D, C)):
                gather_start(i)
            for j in range(C):
                i = j + D
                if i < C:
                    if i - NBUF >= 0:
                        scatter_wait(i - NBUF)
                    gather_start(i)
                gather_wait(j)
                scatter_start(j)
            for j in range(max(0, C - NBUF), C):
                scatter_wait(j)
            return 0

        lax.fori_loop(0, TB // C, chunk, 0)

    @pl.when(c == 0)
    def _():
        run(x_hbm)

    @pl.when(c == 1)
    def _():
        run(y_hbm)

    plsc.subcore_barrier()
    pltpu.sync_copy(acc_sh.at[pl.ds(s * NPT, NPT)],
                    out_hbm.at[c, pl.ds(s * NPT, NPT)])


_scatter_call = pl.kernel(
    _scatter_body,
    out_type=jax.ShapeDtypeStruct((2, N_PAD, F), jnp.bfloat16),
    mesh=_MESH,
    compiler_params=pltpu.CompilerParams(use_tc_tiling_on_sc=False),
    scratch_types=[
        pltpu.VMEM((C, B), jnp.int32),
        pltpu.VMEM((C, B), jnp.int32),
        pltpu.VMEM((NBUF, B, F), jnp.bfloat16),
        pltpu.VMEM_SHARED((N_PAD, F), jnp.bfloat16),
        pltpu.SemaphoreType.DMA,
        pltpu.SemaphoreType.DMA,
    ],
)


_BN = 512


def _scale_body(x_ref, hc_ref, y_ref, xb_ref):
    deg = hc_ref[...] + 1.0
    dinv = 1.0 / jnp.sqrt(deg)
    y_ref[...] = (x_ref[...] * dinv).astype(jnp.bfloat16)
    xb_ref[...] = x_ref[...].astype(jnp.bfloat16)


def _scale_call(x, hc_t):
    grid = (N_PAD // _BN,)
    return pl.pallas_call(
        _scale_body,
        grid=grid,
        in_specs=[
            pl.BlockSpec((_BN, F), lambda i: (i, 0)),
            pl.BlockSpec((_BN, 1), lambda i: (i, 0)),
        ],
        out_specs=[
            pl.BlockSpec((_BN, F), lambda i: (i, 0)),
            pl.BlockSpec((_BN, F), lambda i: (i, 0)),
        ],
        out_shape=[
            jax.ShapeDtypeStruct((N_PAD, F), jnp.bfloat16),
            jax.ShapeDtypeStruct((N_PAD, F), jnp.bfloat16),
        ],
    )(x, hc_t)


def _mm(a, b):
    return jnp.dot(a, b, precision=lax.Precision.HIGHEST,
                   preferred_element_type=jnp.float32)


def _leaky(v):
    return jnp.where(v >= 0, v, 0.01 * v)


def _dense_body(head_ref, x_ref, nbs_ref, s_ref, hr_ref, hc_ref,
                g1t_ref, g2t_ref, b1t_ref, b2t_ref, wt_ref, r_ref, b_ref,
                hk_ref, out_ref):
    x = x_ref[...]
    nn = hr_ref[...]
    nb = nbs_ref[0].astype(jnp.float32) / jnp.maximum(nn, 1.0)
    gamma = _leaky(_mm(x, g1t_ref[...]) + _mm(nb, g2t_ref[...])) + 1.0
    beta = _leaky(_mm(x, b1t_ref[...]) + _mm(nb, b2t_ref[...]))
    out_rel = x + (gamma * r_ref[...] + beta) - nb
    out_ref[...] = out_rel
    dinv = 1.0 / jnp.sqrt(hc_ref[...] + 1.0)
    t = dinv * s_ref[0].astype(jnp.float32) + (dinv * dinv) * x
    h_conv = _mm(t, wt_ref[...]) + b_ref[...]
    h_s = _mm(out_rel, wt_ref[...])
    h_nohead = (h_conv + h_s) / (nn + 1.0)
    hk_ref[...] = jnp.where(head_ref[0, 0] != 0, h_conv, h_nohead)


def _dense_call(head_arr, x, accs, hr_t, hc_t,
                g1t, g2t, b1t, b2t, wt, r, b2d):
    grid = (N_PAD // _BN,)
    blk = lambda i: (i, 0)
    cst = lambda i: (0, 0)
    return pl.pallas_call(
        _dense_body,
        grid=grid,
        in_specs=[
            pl.BlockSpec(memory_space=pltpu.SMEM),
            pl.BlockSpec((_BN, F), blk),
            pl.BlockSpec((1, _BN, F), lambda i: (0, i, 0)),
            pl.BlockSpec((1, _BN, F), lambda i: (1, i, 0)),
            pl.BlockSpec((_BN, 1), blk),
            pl.BlockSpec((_BN, 1), blk),
            pl.BlockSpec((F, F), cst),
            pl.BlockSpec((F, F), cst),
            pl.BlockSpec((F, F), cst),
            pl.BlockSpec((F, F), cst),
            pl.BlockSpec((F, F), cst),
            pl.BlockSpec((1, F), cst),
            pl.BlockSpec((1, F), cst),
        ],
        out_specs=[
            pl.BlockSpec((_BN, F), blk),
            pl.BlockSpec((_BN, F), blk),
        ],
        out_shape=[
            jax.ShapeDtypeStruct((N, F), jnp.float32),
            jax.ShapeDtypeStruct((N, F), jnp.float32),
        ],
    )(head_arr, x, accs, accs, hr_t, hc_t, g1t, g2t, b1t, b2t, wt, r, b2d)


def kernel(x, edge_index, head, G1, G2, B1, B2, r, W_gc, b_gc):
    row = edge_index[0]
    col = edge_index[1]

    pad = E_PAD - E
    gidx = jnp.stack([col, row])
    gidx = jnp.pad(gidx, ((0, 0), (0, pad))).reshape(2, NS, TB, B)
    sidx = jnp.stack([row, col])
    sidx = jnp.pad(sidx, ((0, 0), (0, pad)), constant_values=N)
    sidx = sidx.reshape(2, NS, TB, B)

    hists = _hist_call(sidx)
    hr_t = hists[0].reshape(N_PAD, 1)
    hc_t = hists[1].reshape(N_PAD, 1)

    y_b, x_b = _scale_call(x, hc_t)

    zeros = jnp.zeros((N_PAD, F), jnp.bfloat16)
    accs = _scatter_call(x_b, y_b, gidx, sidx, zeros)

    head_arr = jnp.asarray(head, jnp.int32).reshape(1, 1)
    return _dense_call(head_arr, x, accs, hr_t, hc_t,
                       G1.T, G2.T, B1.T, B2.T, W_gc.T, r,
                       b_gc.reshape(1, F))


# speedup vs baseline: 17.9227x; 1.0098x over previous
import functools

import jax
import jax.numpy as jnp
from jax import lax
from jax.experimental import pallas as pl
from jax.experimental.pallas import tpu as pltpu
from jax.experimental.pallas import tpu_sc as plsc

N = 10000
E = 320000
F = 128

NC = 2
NS = 16
L = 16

N_PAD = 10240
NPT = N_PAD // NS
B = 128
C = 32
NBUF = 4
TB = 160
NBLK = NS * TB
E_PAD = NBLK * B
HB = 128
HTB = E_PAD // (NS * HB)

_MESH = plsc.VectorSubcoreMesh(core_axis_name="c", subcore_axis_name="s")


def _hist_body(idx_hbm, out_hbm, idx_v, ones_v, zero_v, hist_sh, hsem):
    c = lax.axis_index("c")
    s = lax.axis_index("s")

    def zfill(i, _):
        zero_v[pl.ds(i * L, L)] = jnp.zeros((L,), jnp.float32)
        return 0

    lax.fori_loop(0, NPT // L, zfill, 0)

    def ofill(i, _):
        ones_v[pl.ds(i * L, L)] = jnp.full((L,), 1.0, jnp.float32)
        return 0

    lax.fori_loop(0, HB // L, ofill, 0)

    pltpu.sync_copy(zero_v, hist_sh.at[pl.ds(s * NPT, NPT)])
    pltpu.sync_copy(idx_hbm.at[c, s], idx_v)
    plsc.subcore_barrier()

    def body(j, _):
        pltpu.async_copy(ones_v, hist_sh.at[idx_v.at[j]], hsem, add=True)
        return 0

    lax.fori_loop(0, HTB, body, 0)

    def drain(j, _):
        pltpu.make_async_copy(ones_v, hist_sh.at[idx_v.at[j]], hsem).wait()
        return 0

    lax.fori_loop(0, HTB, drain, 0)
    plsc.subcore_barrier()
    pltpu.sync_copy(hist_sh.at[pl.ds(s * NPT, NPT)],
                    out_hbm.at[c, pl.ds(s * NPT, NPT)])


_hist_call = pl.kernel(
    _hist_body,
    out_type=jax.ShapeDtypeStruct((2, N_PAD), jnp.float32),
    mesh=_MESH,
    scratch_types=[
        pltpu.VMEM((HTB, HB), jnp.int32),
        pltpu.VMEM((HB,), jnp.float32),
        pltpu.VMEM((NPT,), jnp.float32),
        pltpu.VMEM_SHARED((N_PAD,), jnp.float32),
        pltpu.SemaphoreType.DMA,
    ],
)


def _scatter_body(x_hbm, y_hbm, gidx_hbm, sidx_hbm, zeros_hbm, out_hbm,
                  gidx_v, sidx_v, rows_v, acc_sh, gsem, ssem):
    c = lax.axis_index("c")
    s = lax.axis_index("s")

    pltpu.sync_copy(zeros_hbm.at[pl.ds(s * NPT, NPT)],
                    acc_sh.at[pl.ds(s * NPT, NPT)])
    plsc.subcore_barrier()

    def run(table):
        def gather_start(j):
            pltpu.async_copy(table.at[gidx_v.at[j]], rows_v.at[j % NBUF],
                             gsem)

        def gather_wait(j):
            pltpu.make_async_copy(table.at[gidx_v.at[j]],
                                  rows_v.at[j % NBUF], gsem).wait()

        def scatter_start(j):
            pltpu.async_copy(rows_v.at[j % NBUF], acc_sh.at[sidx_v.at[j]],
                             ssem, add=True)

        def scatter_wait(j):
            pltpu.make_async_copy(rows_v.at[j % NBUF],
                                  acc_sh.at[sidx_v.at[j]], ssem).wait()

        def chunk(k, _):
            pltpu.sync_copy(gidx_hbm.at[c, s, pl.ds(k * C, C)], gidx_v)
            pltpu.sync_copy(sidx_hbm.at[c, s, pl.ds(k * C, C)], sidx_v)
            D = NBUF - 1
            for i in range(min(D, C)):
                gather_start(i)
            for j in range(C):
                i = j + D
                if i < C:
                    if i - NBUF >= 0:
                        scatter_wait(i - NBUF)
                    gather_start(i)
                gather_wait(j)
                scatter_start(j)
            for j in range(max(0, C - NBUF), C):
                scatter_wait(j)
            return 0

        lax.fori_loop(0, TB // C, chunk, 0)

    @pl.when(c == 0)
    def _():
        run(x_hbm)

    @pl.when(c == 1)
    def _():
        run(y_hbm)

    plsc.subcore_barrier()
    pltpu.sync_copy(acc_sh.at[pl.ds(s * NPT, NPT)],
                    out_hbm.at[c, pl.ds(s * NPT, NPT)])


_scatter_call = pl.kernel(
    _scatter_body,
    out_type=jax.ShapeDtypeStruct((2, N_PAD, F), jnp.bfloat16),
    mesh=_MESH,
    compiler_params=pltpu.CompilerParams(use_tc_tiling_on_sc=False),
    scratch_types=[
        pltpu.VMEM((C, B), jnp.int32),
        pltpu.VMEM((C, B), jnp.int32),
        pltpu.VMEM((NBUF, B, F), jnp.bfloat16),
        pltpu.VMEM_SHARED((N_PAD, F), jnp.bfloat16),
        pltpu.SemaphoreType.DMA,
        pltpu.SemaphoreType.DMA,
    ],
)


_BN = 512


def _scale_body(x_ref, hc_ref, y_ref, xb_ref):
    deg = hc_ref[...] + 1.0
    dinv = 1.0 / jnp.sqrt(deg)
    y_ref[...] = (x_ref[...] * dinv).astype(jnp.bfloat16)
    xb_ref[...] = x_ref[...].astype(jnp.bfloat16)


def _scale_call(x, hc_t):
    grid = (N_PAD // _BN,)
    return pl.pallas_call(
        _scale_body,
        grid=grid,
        in_specs=[
            pl.BlockSpec((_BN, F), lambda i: (i, 0)),
            pl.BlockSpec((_BN, 1), lambda i: (i, 0)),
        ],
        out_specs=[
            pl.BlockSpec((_BN, F), lambda i: (i, 0)),
            pl.BlockSpec((_BN, F), lambda i: (i, 0)),
        ],
        out_shape=[
            jax.ShapeDtypeStruct((N_PAD, F), jnp.bfloat16),
            jax.ShapeDtypeStruct((N_PAD, F), jnp.bfloat16),
        ],
    )(x, hc_t)


def _mm(a, b):
    return jnp.dot(a, b, precision=lax.Precision.HIGHEST,
                   preferred_element_type=jnp.float32)


def _leaky(v):
    return jnp.where(v >= 0, v, 0.01 * v)


def _dense_body(head_ref, x_ref, nbs_ref, s_ref, hr_ref, hc_ref,
                g1t_ref, g2t_ref, b1t_ref, b2t_ref, wt_ref, r_ref, b_ref,
                hk_ref, out_ref):
    x = x_ref[...]
    nn = hr_ref[...]
    nb = nbs_ref[0].astype(jnp.float32) / jnp.maximum(nn, 1.0)
    gamma = _leaky(_mm(x, g1t_ref[...]) + _mm(nb, g2t_ref[...])) + 1.0
    beta = _leaky(_mm(x, b1t_ref[...]) + _mm(nb, b2t_ref[...]))
    out_rel = x + (gamma * r_ref[...] + beta) - nb
    out_ref[...] = out_rel
    dinv = 1.0 / jnp.sqrt(hc_ref[...] + 1.0)
    t = dinv * s_ref[0].astype(jnp.float32) + (dinv * dinv) * x
    h_conv = _mm(t, wt_ref[...]) + b_ref[...]
    h_s = _mm(out_rel, wt_ref[...])
    h_nohead = (h_conv + h_s) / (nn + 1.0)
    hk_ref[...] = jnp.where(head_ref[0, 0] != 0, h_conv, h_nohead)


def _dense_call(head_arr, x, accs, hr_t, hc_t,
                g1t, g2t, b1t, b2t, wt, r, b2d):
    grid = (N_PAD // _BN,)
    blk = lambda i: (i, 0)
    cst = lambda i: (0, 0)
    return pl.pallas_call(
        _dense_body,
        grid=grid,
        in_specs=[
            pl.BlockSpec(memory_space=pltpu.SMEM),
            pl.BlockSpec((_BN, F), blk),
            pl.BlockSpec((1, _BN, F), lambda i: (0, i, 0)),
            pl.BlockSpec((1, _BN, F), lambda i: (1, i, 0)),
            pl.BlockSpec((_BN, 1), blk),
            pl.BlockSpec((_BN, 1), blk),
            pl.BlockSpec((F, F), cst),
            pl.BlockSpec((F, F), cst),
            pl.BlockSpec((F, F), cst),
            pl.BlockSpec((F, F), cst),
            pl.BlockSpec((F, F), cst),
            pl.BlockSpec((1, F), cst),
            pl.BlockSpec((1, F), cst),
        ],
        out_specs=[
            pl.BlockSpec((_BN, F), blk),
            pl.BlockSpec((_BN, F), blk),
        ],
        out_shape=[
            jax.ShapeDtypeStruct((N, F), jnp.float32),
            jax.ShapeDtypeStruct((N, F), jnp.float32),
        ],
    )(head_arr, x, accs, accs, hr_t, hc_t, g1t, g2t, b1t, b2t, wt, r, b2d)


def kernel(x, edge_index, head, G1, G2, B1, B2, r, W_gc, b_gc):
    row = edge_index[0]
    col = edge_index[1]

    pad = E_PAD - E
    gidx = jnp.stack([col, row])
    gidx = jnp.pad(gidx, ((0, 0), (0, pad))).reshape(2, NS, TB, B)
    sidx = jnp.stack([row, col])
    sidx = jnp.pad(sidx, ((0, 0), (0, pad)), constant_values=N)
    sidx = sidx.reshape(2, NS, TB, B)

    hists = _hist_call(sidx)
    hr_t = hists[0].reshape(N_PAD, 1)
    hc_t = hists[1].reshape(N_PAD, 1)

    y_b, x_b = _scale_call(x, hc_t)

    zeros = jnp.zeros((N_PAD, F), jnp.bfloat16)
    accs = _scatter_call(x_b, y_b, gidx, sidx, zeros)

    head_arr = jnp.asarray(head, jnp.int32).reshape(1, 1)
    return _dense_call(head_arr, x, accs, hr_t, hc_t,
                       G1.T, G2.T, B1.T, B2.T, W_gc.T, r,
                       b_gc.reshape(1, F))
